# baseline (speedup 1.0000x reference)
import functools

import jax
import jax.numpy as jnp
from jax.experimental import pallas as pl
from jax.experimental.pallas import tpu as pltpu


def _gat_banded_kernel(x_ref, wl_ref, cl_ref, wr_ref, cr_ref, wy_ref, cy_ref,
                       arep_ref, seg_ref, bfc_ref, out_ref, *, n_nodes,
                       halves):
    rows = x_ref.shape[0]
    hr = rows // halves

    for hh in range(halves):
        base = hh * hr
        g = hr // n_nodes
        x = x_ref[base:base + hr, :]

        cl = jnp.tile(cl_ref[...], (g, 1))
        cr = jnp.tile(cr_ref[...], (g, 1))
        cy = jnp.tile(cy_ref[...], (g, 1))
        xl = jnp.dot(x, wl_ref[...], preferred_element_type=jnp.float32) + cl
        xr_b = (jnp.dot(x, wr_ref[...], preferred_element_type=jnp.float32)
                + cr).astype(jnp.bfloat16)
        y = (jnp.dot(x, wy_ref[...], preferred_element_type=jnp.float32)
             + cy).astype(jnp.bfloat16)

        xlm = pltpu.roll(xl, 1, axis=0)
        xlp = pltpu.roll(xl, hr - 1, axis=0)

        xl_b = xl.astype(jnp.bfloat16)
        xlm_b = xlm.astype(jnp.bfloat16)
        xlp_b = xlp.astype(jnp.bfloat16)

        def band(a, b):
            v = a + b
            lr = jnp.maximum(v, 0.2 * v)
            return jax.lax.dot_general(
                arep_ref[...], lr, (((1,), (1,)), ((), ())),
                preferred_element_type=jnp.float32)

        e0 = band(xr_b, xl_b)
        em = band(xr_b, xlm_b)
        ep = band(xr_b, xlp_b)

        t = jax.lax.broadcasted_iota(jnp.int32, (8, hr), 1) & (n_nodes - 1)
        p0 = jnp.exp(e0)
        pm = jnp.where(t == 0, 0.0, jnp.exp(em))
        pp = jnp.where(t == n_nodes - 1, 0.0, jnp.exp(ep))
        r = 1.0 / (p0 + pm + pp)

        am_up = pltpu.roll(pm * r, hr - 1, axis=1)
        ap_dn = pltpu.roll(pp * r, 1, axis=1)
        w8 = p0 * r + am_up + ap_dn

        gc = seg_ref.shape[0]
        rc = gc * n_nodes
        for c in range(g // gc):
            wseg = (jnp.tile(w8[:, c * rc:(c + 1) * rc], (gc // 8, 1))
                    * seg_ref[...]).astype(jnp.bfloat16)
            o0 = hh * (hr // n_nodes) + c * gc
            out_ref[o0:o0 + gc, :] = (
                jnp.dot(wseg, y[c * rc:(c + 1) * rc, :],
                        preferred_element_type=jnp.float32) + bfc_ref[...])


def kernel(x, we_T, pe_be, wl_T, bl, wr_T, br, att, mask, wfc_T, bfc):
    del mask
    b, n, din = x.shape
    h = we_T.shape[1]
    c_pad = wfc_T.shape[1]

    wl_f = jnp.dot(we_T, wl_T, preferred_element_type=jnp.float32)
    cl_f = jnp.dot(pe_be, wl_T, preferred_element_type=jnp.float32) + bl
    wr_f = jnp.dot(we_T, wr_T, preferred_element_type=jnp.float32)
    cr_f = jnp.dot(pe_be, wr_T, preferred_element_type=jnp.float32) + br
    wy_f = jnp.dot(wl_f, wfc_T, preferred_element_type=jnp.float32)
    cy_f = jnp.dot(cl_f, wfc_T, preferred_element_type=jnp.float32)
    arep = jnp.tile(att.reshape(1, h), (8, 1)).astype(jnp.bfloat16)

    graphs_per_block = 256
    while b % graphs_per_block:
        graphs_per_block //= 2
    rows = graphs_per_block * n
    halves = 1
    xf = x.reshape(b * n, din).astype(jnp.bfloat16)

    seg_graphs = min(graphs_per_block // halves, 64)
    seg = (jnp.arange(seg_graphs).reshape(-1, 1) ==
           (jnp.arange(seg_graphs * n) // n).reshape(1, -1)).astype(jnp.float32)

    def fixed(shape):
        nd = len(shape)
        return pl.BlockSpec(shape, lambda i, _nd=nd: (0,) * _nd)

    out = pl.pallas_call(
        functools.partial(_gat_banded_kernel, n_nodes=n, halves=halves),
        grid=(b // graphs_per_block,),
        out_shape=jax.ShapeDtypeStruct((b, c_pad), jnp.float32),
        in_specs=[
            pl.BlockSpec((rows, din), lambda i: (i, 0)),
            fixed((din, h)),
            fixed((n, h)),
            fixed((din, h)),
            fixed((n, h)),
            fixed((din, c_pad)),
            fixed((n, c_pad)),
            fixed((8, h)),
            fixed((seg_graphs, seg_graphs * n)),
            fixed((1, c_pad)),
        ],
        out_specs=pl.BlockSpec((graphs_per_block, c_pad), lambda i: (i, 0)),
        compiler_params=pltpu.CompilerParams(
            dimension_semantics=("parallel",)),
    )(xf, wl_f.astype(jnp.bfloat16), cl_f, wr_f.astype(jnp.bfloat16),
      cr_f, wy_f.astype(jnp.bfloat16), cy_f, arep, seg, bfc)
    return out


# speedup vs baseline: 1.3151x; 1.3151x over previous
import functools

import jax
import jax.numpy as jnp
from jax.experimental import pallas as pl
from jax.experimental.pallas import tpu as pltpu


def _gat_banded_kernel(x_ref, wall_ref, call_ref, arep_ref, seg_ref, bfc_ref,
                       out_ref, *, n_nodes, halves):
    rows = x_ref.shape[0]
    hr = rows // halves
    h = x_ref.shape[1]

    for hh in range(halves):
        base = hh * hr
        g = hr // n_nodes
        x = x_ref[base:base + hr, :].astype(jnp.bfloat16)

        proj = (jnp.dot(x, wall_ref[...], preferred_element_type=jnp.float32)
                + jnp.tile(call_ref[...], (g, 1)))
        xl = proj[:, :h]
        xr_b = proj[:, h:2 * h].astype(jnp.bfloat16)
        y = proj[:, 2 * h:3 * h].astype(jnp.bfloat16)

        xlm = pltpu.roll(xl, 1, axis=0)
        xlp = pltpu.roll(xl, hr - 1, axis=0)

        xl_b = xl.astype(jnp.bfloat16)
        xlm_b = xlm.astype(jnp.bfloat16)
        xlp_b = xlp.astype(jnp.bfloat16)

        def band(a, b):
            v = a + b
            lr = jnp.maximum(v, 0.2 * v)
            return jax.lax.dot_general(
                arep_ref[...], lr, (((1,), (1,)), ((), ())),
                preferred_element_type=jnp.float32)

        e0 = band(xr_b, xl_b)
        em = band(xr_b, xlm_b)
        ep = band(xr_b, xlp_b)

        t = jax.lax.broadcasted_iota(jnp.int32, (8, hr), 1) & (n_nodes - 1)
        p0 = jnp.exp(e0)
        pm = jnp.where(t == 0, 0.0, jnp.exp(em))
        pp = jnp.where(t == n_nodes - 1, 0.0, jnp.exp(ep))
        r = 1.0 / (p0 + pm + pp)

        am_up = pltpu.roll(pm * r, hr - 1, axis=1)
        ap_dn = pltpu.roll(pp * r, 1, axis=1)
        w8 = p0 * r + am_up + ap_dn

        gc = seg_ref.shape[0]
        rc = gc * n_nodes
        for c in range(g // gc):
            wseg = (jnp.tile(w8[:, c * rc:(c + 1) * rc], (gc // 8, 1))
                    * seg_ref[...]).astype(jnp.bfloat16)
            o0 = hh * (hr // n_nodes) + c * gc
            out_ref[o0:o0 + gc, :] = (
                jnp.dot(wseg, y[c * rc:(c + 1) * rc, :],
                        preferred_element_type=jnp.float32) + bfc_ref[...])


def kernel(x, we_T, pe_be, wl_T, bl, wr_T, br, att, mask, wfc_T, bfc):
    del mask
    b, n, din = x.shape
    h = we_T.shape[1]
    c_pad = wfc_T.shape[1]

    wl_f = jnp.dot(we_T, wl_T, preferred_element_type=jnp.float32)
    cl_f = jnp.dot(pe_be, wl_T, preferred_element_type=jnp.float32) + bl
    wr_f = jnp.dot(we_T, wr_T, preferred_element_type=jnp.float32)
    cr_f = jnp.dot(pe_be, wr_T, preferred_element_type=jnp.float32) + br
    wy_f = jnp.dot(wl_f, wfc_T, preferred_element_type=jnp.float32)
    cy_f = jnp.dot(cl_f, wfc_T, preferred_element_type=jnp.float32)
    wall = jnp.concatenate([wl_f, wr_f, wy_f], axis=1).astype(jnp.bfloat16)
    call = jnp.concatenate([cl_f, cr_f, cy_f], axis=1)
    arep = jnp.tile(att.reshape(1, h), (8, 1)).astype(jnp.bfloat16)

    graphs_per_block = 256
    while b % graphs_per_block:
        graphs_per_block //= 2
    rows = graphs_per_block * n
    halves = 1
    xf = x.reshape(b * n, din)

    seg_graphs = min(graphs_per_block // halves, 64)
    seg = (jnp.arange(seg_graphs).reshape(-1, 1) ==
           (jnp.arange(seg_graphs * n) // n).reshape(1, -1)).astype(jnp.float32)

    def fixed(shape):
        nd = len(shape)
        return pl.BlockSpec(shape, lambda i, _nd=nd: (0,) * _nd)

    out = pl.pallas_call(
        functools.partial(_gat_banded_kernel, n_nodes=n, halves=halves),
        grid=(b // graphs_per_block,),
        out_shape=jax.ShapeDtypeStruct((b, c_pad), jnp.float32),
        in_specs=[
            pl.BlockSpec((rows, din), lambda i: (i, 0)),
            fixed((din, h + h + c_pad)),
            fixed((n, h + h + c_pad)),
            fixed((8, h)),
            fixed((seg_graphs, seg_graphs * n)),
            fixed((1, c_pad)),
        ],
        out_specs=pl.BlockSpec((graphs_per_block, c_pad), lambda i: (i, 0)),
        compiler_params=pltpu.CompilerParams(
            dimension_semantics=("parallel",)),
    )(xf, wall, call, arep, seg, bfc)
    return out


# speedup vs baseline: 1.3548x; 1.0302x over previous
import functools

import jax
import jax.numpy as jnp
from jax.experimental import pallas as pl
from jax.experimental.pallas import tpu as pltpu


def _gat_banded_kernel(x_ref, wl_ref, cl_ref, wr_ref, cr_ref, wy_ref, cy_ref,
                       arep_ref, seg_ref, bfc_ref, out_ref, *, n_nodes,
                       halves):
    rows = x_ref.shape[0]
    hr = rows // halves

    for hh in range(halves):
        base = hh * hr
        g = hr // n_nodes
        x = x_ref[base:base + hr, :].astype(jnp.bfloat16)

        cl = jnp.tile(cl_ref[...], (g, 1))
        cr = jnp.tile(cr_ref[...], (g, 1))
        cy = jnp.tile(cy_ref[...], (g, 1))
        xl = jnp.dot(x, wl_ref[...], preferred_element_type=jnp.float32) + cl
        xr_b = (jnp.dot(x, wr_ref[...], preferred_element_type=jnp.float32)
                + cr).astype(jnp.bfloat16)
        y = (jnp.dot(x, wy_ref[...], preferred_element_type=jnp.float32)
             + cy).astype(jnp.bfloat16)

        xlm = pltpu.roll(xl, 1, axis=0)
        xlp = pltpu.roll(xl, hr - 1, axis=0)

        xl_b = xl.astype(jnp.bfloat16)
        xlm_b = xlm.astype(jnp.bfloat16)
        xlp_b = xlp.astype(jnp.bfloat16)

        def band(a, b):
            v = a + b
            lr = jnp.maximum(v, 0.2 * v)
            return jax.lax.dot_general(
                arep_ref[...], lr, (((1,), (1,)), ((), ())),
                preferred_element_type=jnp.float32)

        e0 = band(xr_b, xl_b)
        em = band(xr_b, xlm_b)
        ep = band(xr_b, xlp_b)

        t = jax.lax.broadcasted_iota(jnp.int32, (8, hr), 1) & (n_nodes - 1)
        p0 = jnp.exp(e0)
        pm = jnp.where(t == 0, 0.0, jnp.exp(em))
        pp = jnp.where(t == n_nodes - 1, 0.0, jnp.exp(ep))
        r = 1.0 / (p0 + pm + pp)

        am_up = pltpu.roll(pm * r, hr - 1, axis=1)
        ap_dn = pltpu.roll(pp * r, 1, axis=1)
        w8 = p0 * r + am_up + ap_dn

        gc = seg_ref.shape[0]
        rc = gc * n_nodes
        for c in range(g // gc):
            wseg = (jnp.tile(w8[:, c * rc:(c + 1) * rc], (gc // 8, 1))
                    * seg_ref[...]).astype(jnp.bfloat16)
            o0 = hh * (hr // n_nodes) + c * gc
            out_ref[o0:o0 + gc, :] = (
                jnp.dot(wseg, y[c * rc:(c + 1) * rc, :],
                        preferred_element_type=jnp.float32) + bfc_ref[...])


def kernel(x, we_T, pe_be, wl_T, bl, wr_T, br, att, mask, wfc_T, bfc):
    del mask
    b, n, din = x.shape
    h = we_T.shape[1]
    c_pad = wfc_T.shape[1]

    wl_f = jnp.dot(we_T, wl_T, preferred_element_type=jnp.float32)
    cl_f = jnp.dot(pe_be, wl_T, preferred_element_type=jnp.float32) + bl
    wr_f = jnp.dot(we_T, wr_T, preferred_element_type=jnp.float32)
    cr_f = jnp.dot(pe_be, wr_T, preferred_element_type=jnp.float32) + br
    wy_f = jnp.dot(wl_f, wfc_T, preferred_element_type=jnp.float32)
    cy_f = jnp.dot(cl_f, wfc_T, preferred_element_type=jnp.float32)
    arep = jnp.tile(att.reshape(1, h), (8, 1)).astype(jnp.bfloat16)

    graphs_per_block = 256
    while b % graphs_per_block:
        graphs_per_block //= 2
    rows = graphs_per_block * n
    halves = 1
    xf = x.reshape(b * n, din)

    seg_graphs = min(graphs_per_block // halves, 64)
    seg = (jnp.arange(seg_graphs).reshape(-1, 1) ==
           (jnp.arange(seg_graphs * n) // n).reshape(1, -1)).astype(jnp.float32)

    def fixed(shape):
        nd = len(shape)
        return pl.BlockSpec(shape, lambda i, _nd=nd: (0,) * _nd)

    out = pl.pallas_call(
        functools.partial(_gat_banded_kernel, n_nodes=n, halves=halves),
        grid=(b // graphs_per_block,),
        out_shape=jax.ShapeDtypeStruct((b, c_pad), jnp.float32),
        in_specs=[
            pl.BlockSpec((rows, din), lambda i: (i, 0)),
            fixed((din, h)),
            fixed((n, h)),
            fixed((din, h)),
            fixed((n, h)),
            fixed((din, c_pad)),
            fixed((n, c_pad)),
            fixed((8, h)),
            fixed((seg_graphs, seg_graphs * n)),
            fixed((1, c_pad)),
        ],
        out_specs=pl.BlockSpec((graphs_per_block, c_pad), lambda i: (i, 0)),
        compiler_params=pltpu.CompilerParams(
            dimension_semantics=("parallel",)),
    )(xf, wl_f.astype(jnp.bfloat16), cl_f, wr_f.astype(jnp.bfloat16),
      cr_f, wy_f.astype(jnp.bfloat16), cy_f, arep, seg, bfc)
    return out


# speedup vs baseline: 1.4268x; 1.0531x over previous
import functools

import jax
import jax.numpy as jnp
from jax.experimental import pallas as pl
from jax.experimental.pallas import tpu as pltpu


def _gat_banded_kernel(x_ref, wl_ref, cl_ref, wr_ref, cr_ref, wy_ref, cy_ref,
                       arep_ref, seg_ref, bfc_ref, out_ref, *, n_nodes,
                       halves):
    rows = x_ref.shape[0]
    hr = rows // halves

    for hh in range(halves):
        base = hh * hr
        g = hr // n_nodes
        x = x_ref[base:base + hr, :].astype(jnp.bfloat16)

        cl = jnp.tile(cl_ref[...], (g, 1))
        cr = jnp.tile(cr_ref[...], (g, 1))
        cy = jnp.tile(cy_ref[...], (g, 1))
        xl = jnp.dot(x, wl_ref[...], preferred_element_type=jnp.float32) + cl
        xr_b = (jnp.dot(x, wr_ref[...], preferred_element_type=jnp.float32)
                + cr).astype(jnp.bfloat16)
        y = (jnp.dot(x, wy_ref[...], preferred_element_type=jnp.float32)
             + cy).astype(jnp.bfloat16)

        xlm = pltpu.roll(xl, 1, axis=0)
        xlp = pltpu.roll(xl, hr - 1, axis=0)

        xl_b = xl.astype(jnp.bfloat16)
        xlm_b = xlm.astype(jnp.bfloat16)
        xlp_b = xlp.astype(jnp.bfloat16)

        def band(a, b):
            v = a + b
            lr = jnp.maximum(v, 0.2 * v)
            return jax.lax.dot_general(
                arep_ref[...], lr, (((1,), (1,)), ((), ())),
                preferred_element_type=jnp.float32)

        e0 = band(xr_b, xl_b)
        em = band(xr_b, xlm_b)
        ep = band(xr_b, xlp_b)

        t = jax.lax.broadcasted_iota(jnp.int32, (8, hr), 1) & (n_nodes - 1)
        p0 = jnp.exp(e0)
        pm = jnp.where(t == 0, 0.0, jnp.exp(em))
        pp = jnp.where(t == n_nodes - 1, 0.0, jnp.exp(ep))
        r = 1.0 / (p0 + pm + pp)

        am_up = pltpu.roll(pm * r, hr - 1, axis=1)
        ap_dn = pltpu.roll(pp * r, 1, axis=1)
        w8 = p0 * r + am_up + ap_dn

        gc = seg_ref.shape[0]
        rc = gc * n_nodes
        for c in range(g // gc):
            wseg = (jnp.tile(w8[:, c * rc:(c + 1) * rc], (gc // 8, 1))
                    * seg_ref[...]).astype(jnp.bfloat16)
            o0 = hh * (hr // n_nodes) + c * gc
            out_ref[o0:o0 + gc, :] = (
                jnp.dot(wseg, y[c * rc:(c + 1) * rc, :],
                        preferred_element_type=jnp.float32) + bfc_ref[...])


def kernel(x, we_T, pe_be, wl_T, bl, wr_T, br, att, mask, wfc_T, bfc):
    del mask
    b, n, din = x.shape
    h = we_T.shape[1]
    c_pad = wfc_T.shape[1]

    wl_f = jnp.dot(we_T, wl_T, preferred_element_type=jnp.float32)
    cl_f = jnp.dot(pe_be, wl_T, preferred_element_type=jnp.float32) + bl
    wr_f = jnp.dot(we_T, wr_T, preferred_element_type=jnp.float32)
    cr_f = jnp.dot(pe_be, wr_T, preferred_element_type=jnp.float32) + br
    wy_f = jnp.dot(wl_f, wfc_T, preferred_element_type=jnp.float32)
    cy_f = jnp.dot(cl_f, wfc_T, preferred_element_type=jnp.float32)
    arep = jnp.tile(att.reshape(1, h), (8, 1)).astype(jnp.bfloat16)

    graphs_per_block = 512
    while b % graphs_per_block:
        graphs_per_block //= 2
    rows = graphs_per_block * n
    halves = 1
    xf = x.reshape(b * n, din)

    seg_graphs = min(graphs_per_block // halves, 64)
    seg = (jnp.arange(seg_graphs).reshape(-1, 1) ==
           (jnp.arange(seg_graphs * n) // n).reshape(1, -1)).astype(jnp.float32)

    def fixed(shape):
        nd = len(shape)
        return pl.BlockSpec(shape, lambda i, _nd=nd: (0,) * _nd)

    out = pl.pallas_call(
        functools.partial(_gat_banded_kernel, n_nodes=n, halves=halves),
        grid=(b // graphs_per_block,),
        out_shape=jax.ShapeDtypeStruct((b, c_pad), jnp.float32),
        in_specs=[
            pl.BlockSpec((rows, din), lambda i: (i, 0)),
            fixed((din, h)),
            fixed((n, h)),
            fixed((din, h)),
            fixed((n, h)),
            fixed((din, c_pad)),
            fixed((n, c_pad)),
            fixed((8, h)),
            fixed((seg_graphs, seg_graphs * n)),
            fixed((1, c_pad)),
        ],
        out_specs=pl.BlockSpec((graphs_per_block, c_pad), lambda i: (i, 0)),
        compiler_params=pltpu.CompilerParams(
            dimension_semantics=("parallel",)),
    )(xf, wl_f.astype(jnp.bfloat16), cl_f, wr_f.astype(jnp.bfloat16),
      cr_f, wy_f.astype(jnp.bfloat16), cy_f, arep, seg, bfc)
    return out


# speedup vs baseline: 1.4340x; 1.0050x over previous
import functools

import jax
import jax.numpy as jnp
from jax.experimental import pallas as pl
from jax.experimental.pallas import tpu as pltpu


def _gat_banded_kernel(x_ref, wl_ref, cl_ref, wr_ref, cr_ref, wy_ref, cy_ref,
                       arep_ref, seg_ref, bfc_ref, out_ref, *, n_nodes,
                       halves):
    rows = x_ref.shape[0]
    hr = rows // halves

    for hh in range(halves):
        base = hh * hr
        g = hr // n_nodes
        x = x_ref[base:base + hr, :].astype(jnp.bfloat16)

        cl = jnp.tile(cl_ref[...], (g, 1))
        cr = jnp.tile(cr_ref[...], (g, 1))
        cy = jnp.tile(cy_ref[...], (g, 1))
        xl = jnp.dot(x, wl_ref[...], preferred_element_type=jnp.float32) + cl
        xr_b = (jnp.dot(x, wr_ref[...], preferred_element_type=jnp.float32)
                + cr).astype(jnp.bfloat16)
        y = (jnp.dot(x, wy_ref[...], preferred_element_type=jnp.float32)
             + cy).astype(jnp.bfloat16)

        xlm = pltpu.roll(xl, 1, axis=0)
        xlp = pltpu.roll(xl, hr - 1, axis=0)

        xl_b = xl.astype(jnp.bfloat16)
        xlm_b = xlm.astype(jnp.bfloat16)
        xlp_b = xlp.astype(jnp.bfloat16)

        def band(a, b):
            v = a + b
            lr = jnp.maximum(v, 0.2 * v)
            return jax.lax.dot_general(
                arep_ref[...], lr, (((1,), (1,)), ((), ())),
                preferred_element_type=jnp.float32)

        e0 = band(xr_b, xl_b)
        em = band(xr_b, xlm_b)
        ep = band(xr_b, xlp_b)

        t = jax.lax.broadcasted_iota(jnp.int32, (8, hr), 1) & (n_nodes - 1)
        p0 = jnp.exp(e0)
        pm = jnp.where(t == 0, 0.0, jnp.exp(em))
        pp = jnp.where(t == n_nodes - 1, 0.0, jnp.exp(ep))
        r = 1.0 / (p0 + pm + pp)

        am_up = pltpu.roll(pm * r, hr - 1, axis=1)
        ap_dn = pltpu.roll(pp * r, 1, axis=1)
        w8 = p0 * r + am_up + ap_dn

        gc = seg_ref.shape[0]
        rc = gc * n_nodes
        for c in range(g // gc):
            wseg = (jnp.tile(w8[:, c * rc:(c + 1) * rc], (gc // 8, 1))
                    * seg_ref[...]).astype(jnp.bfloat16)
            o0 = hh * (hr // n_nodes) + c * gc
            out_ref[o0:o0 + gc, :] = (
                jnp.dot(wseg, y[c * rc:(c + 1) * rc, :],
                        preferred_element_type=jnp.float32) + bfc_ref[...])


def kernel(x, we_T, pe_be, wl_T, bl, wr_T, br, att, mask, wfc_T, bfc):
    del mask
    b, n, din = x.shape
    h = we_T.shape[1]
    c_pad = wfc_T.shape[1]

    wl_f = jnp.dot(we_T, wl_T, preferred_element_type=jnp.float32)
    cl_f = jnp.dot(pe_be, wl_T, preferred_element_type=jnp.float32) + bl
    wr_f = jnp.dot(we_T, wr_T, preferred_element_type=jnp.float32)
    cr_f = jnp.dot(pe_be, wr_T, preferred_element_type=jnp.float32) + br
    wy_f = jnp.dot(wl_f, wfc_T, preferred_element_type=jnp.float32)
    cy_f = jnp.dot(cl_f, wfc_T, preferred_element_type=jnp.float32)
    arep = jnp.tile(att.reshape(1, h), (8, 1)).astype(jnp.bfloat16)

    graphs_per_block = 1024
    while b % graphs_per_block:
        graphs_per_block //= 2
    rows = graphs_per_block * n
    halves = 1
    xf = x.reshape(b * n, din)

    seg_graphs = min(graphs_per_block // halves, 64)
    seg = (jnp.arange(seg_graphs).reshape(-1, 1) ==
           (jnp.arange(seg_graphs * n) // n).reshape(1, -1)).astype(jnp.float32)

    def fixed(shape):
        nd = len(shape)
        return pl.BlockSpec(shape, lambda i, _nd=nd: (0,) * _nd)

    out = pl.pallas_call(
        functools.partial(_gat_banded_kernel, n_nodes=n, halves=halves),
        grid=(b // graphs_per_block,),
        out_shape=jax.ShapeDtypeStruct((b, c_pad), jnp.float32),
        in_specs=[
            pl.BlockSpec((rows, din), lambda i: (i, 0)),
            fixed((din, h)),
            fixed((n, h)),
            fixed((din, h)),
            fixed((n, h)),
            fixed((din, c_pad)),
            fixed((n, c_pad)),
            fixed((8, h)),
            fixed((seg_graphs, seg_graphs * n)),
            fixed((1, c_pad)),
        ],
        out_specs=pl.BlockSpec((graphs_per_block, c_pad), lambda i: (i, 0)),
        compiler_params=pltpu.CompilerParams(
            dimension_semantics=("parallel",)),
    )(xf, wl_f.astype(jnp.bfloat16), cl_f, wr_f.astype(jnp.bfloat16),
      cr_f, wy_f.astype(jnp.bfloat16), cy_f, arep, seg, bfc)
    return out
